# Initial kernel scaffold; baseline (speedup 1.0000x reference)
import jax
import jax.numpy as jnp
from jax.experimental import pallas as pl


def kernel(inputs, emb_table, pos_table):
    raise NotImplementedError("write your pallas kernel here")


# speedup vs baseline: 2.8184x; 2.8184x over previous
import functools
import math

import jax
import jax.numpy as jnp
from jax import lax
from jax.experimental import pallas as pl
from jax.experimental.pallas import tpu as pltpu
from jax.experimental.pallas import tpu_sc as plsc

D = 64
T = 200
NC, NS = 2, 16
NW = NC * NS
CHUNK = 400
_SLICES = [(o, min(128, CHUNK - o)) for o in range(0, CHUNK, 128)]


def _make_emb_kernel(n_rows: int):
    rows_w = n_rows // NW
    n_chunks = rows_w // CHUNK
    mesh = plsc.VectorSubcoreMesh(
        core_axis_name="c", subcore_axis_name="s",
        num_cores=NC, num_subcores=NS)

    @functools.partial(
        pl.kernel,
        out_type=jax.ShapeDtypeStruct((n_rows, D), jnp.float32),
        mesh=mesh,
        scratch_types=[
            pltpu.VMEM((CHUNK,), jnp.int32),
            pltpu.VMEM((CHUNK, D), jnp.float32),
            pltpu.VMEM((CHUNK, D), jnp.float32),
            pltpu.SemaphoreType.DMA,
        ],
        compiler_params=pltpu.CompilerParams(use_tc_tiling_on_sc=False),
    )
    def emb_kernel(idx_hbm, table_hbm, pos_hbm, out_hbm,
                   idx_v, rows_v, pos_v, sem):
        wid = lax.axis_index("s") * NC + lax.axis_index("c")
        base = wid * rows_w
        scale = jnp.float32(math.sqrt(D))
        for rep in range(CHUNK // T):
            pltpu.sync_copy(pos_hbm, pos_v.at[pl.ds(rep * T, T)])

        def chunk_body(c, carry):
            start = base + c * CHUNK
            pltpu.sync_copy(idx_hbm.at[pl.ds(start, CHUNK)], idx_v)
            copies = [
                pltpu.async_copy(table_hbm.at[idx_v.at[pl.ds(o, s)]],
                                 rows_v.at[pl.ds(o, s)], sem)
                for o, s in _SLICES
            ]
            for cp in copies:
                cp.wait()

            def row_body(r, rc):
                for k in range(D // 16):
                    sl = pl.ds(k * 16, 16)
                    rows_v[r, sl] = rows_v[r, sl] * scale + pos_v[r, sl]
                return rc
            lax.fori_loop(0, CHUNK, row_body, 0, unroll=2)
            pltpu.sync_copy(rows_v, out_hbm.at[pl.ds(start, CHUNK)])
            return carry
        lax.fori_loop(0, n_chunks, chunk_body, 0)

    return emb_kernel


def kernel(inputs, emb_table, pos_table):
    b, t = inputs.shape
    n_rows = b * t
    assert t == T and n_rows % (NW * CHUNK) == 0
    idx = jnp.reshape(inputs, (n_rows,)).astype(jnp.int32)
    out = _make_emb_kernel(n_rows)(idx, emb_table, pos_table)
    return jnp.reshape(out, (b, t, D))
